# Initial kernel scaffold; baseline (speedup 1.0000x reference)
import jax
import jax.numpy as jnp
from jax.experimental import pallas as pl


def kernel(feat, edge_index, p, s_f, W, attn_l, attn_r, bias):
    raise NotImplementedError("write your pallas kernel here")


# speedup vs baseline: 24.4611x; 24.4611x over previous
import functools

import jax
import jax.numpy as jnp
from jax import lax
from jax.experimental import pallas as pl
from jax.experimental.pallas import tpu as pltpu
from jax.experimental.pallas import tpu_sc as plsc

N = 10000
E = 320000
F = 128

NC = 2
NS = 16
L = 16
NW = NC * NS
ET = E // NW
SCH = 80
NB = ET // SCH
EC = 2000
NCH = ET // EC
NCB = EC // SCH
NP = 10240
RPT = NP // NS

BN = 1000


def _k1_body(feat_ref, wt_ref, al_ref, ar_ref,
             fs_ref, el_ref, er_ref, mu_ref, ml_ref, mr_ref):
    i = pl.program_id(0)
    fs = jnp.dot(feat_ref[...], wt_ref[...], preferred_element_type=jnp.float32)
    fs_ref[...] = fs
    el = jnp.sum(fs * al_ref[...], axis=1)
    er = jnp.sum(fs * ar_ref[...], axis=1)
    el_ref[...] = el[:, None]
    er_ref[...] = er[:, None]
    bmin = jnp.min(fs)
    bml = jnp.max(el)
    bmr = jnp.max(er)

    @pl.when(i == 0)
    def _():
        mu_ref[0, 0] = bmin
        ml_ref[0, 0] = bml
        mr_ref[0, 0] = bmr

    @pl.when(i > 0)
    def _():
        mu_ref[0, 0] = jnp.minimum(mu_ref[0, 0], bmin)
        ml_ref[0, 0] = jnp.maximum(ml_ref[0, 0], bml)
        mr_ref[0, 0] = jnp.maximum(mr_ref[0, 0], bmr)


_k1 = pl.pallas_call(
    _k1_body,
    grid=(N // BN,),
    in_specs=[
        pl.BlockSpec((BN, F), lambda i: (i, 0)),
        pl.BlockSpec((F, F), lambda i: (0, 0)),
        pl.BlockSpec((1, F), lambda i: (0, 0)),
        pl.BlockSpec((1, F), lambda i: (0, 0)),
    ],
    out_specs=[
        pl.BlockSpec((BN, F), lambda i: (i, 0)),
        pl.BlockSpec((BN, 1), lambda i: (i, 0)),
        pl.BlockSpec((BN, 1), lambda i: (i, 0)),
        pl.BlockSpec((1, 1), lambda i: (0, 0), memory_space=pltpu.SMEM),
        pl.BlockSpec((1, 1), lambda i: (0, 0), memory_space=pltpu.SMEM),
        pl.BlockSpec((1, 1), lambda i: (0, 0), memory_space=pltpu.SMEM),
    ],
    out_shape=[
        jax.ShapeDtypeStruct((N, F), jnp.float32),
        jax.ShapeDtypeStruct((N, 1), jnp.float32),
        jax.ShapeDtypeStruct((N, 1), jnp.float32),
        jax.ShapeDtypeStruct((1, 1), jnp.float32),
        jax.ShapeDtypeStruct((1, 1), jnp.float32),
        jax.ShapeDtypeStruct((1, 1), jnp.float32),
    ],
)


def _k2_body(fs_ref, d0_ref, d1_ref, mu_ref, p_ref, sf_ref,
             pf_ref, inv_ref):
    p_eff = sf_ref[0, 0] + 1.0 / (1.0 + jnp.exp(-p_ref[0, 0]))
    x = fs_ref[...] - mu_ref[0, 0] + 1e-6
    pf_ref[...] = jnp.exp(p_eff * jnp.log(x))
    inv_ref[...] = 1.0 / (d0_ref[...] + d1_ref[...])


_k2 = pl.pallas_call(
    _k2_body,
    grid=(N // BN,),
    in_specs=[
        pl.BlockSpec((BN, F), lambda i: (i, 0)),
        pl.BlockSpec((BN, 1), lambda i: (i, 0)),
        pl.BlockSpec((BN, 1), lambda i: (i, 0)),
        pl.BlockSpec((1, 1), lambda i: (0, 0), memory_space=pltpu.SMEM),
        pl.BlockSpec((1, 1), lambda i: (0, 0), memory_space=pltpu.SMEM),
        pl.BlockSpec((1, 1), lambda i: (0, 0), memory_space=pltpu.SMEM),
    ],
    out_specs=[
        pl.BlockSpec((BN, F), lambda i: (i, 0)),
        pl.BlockSpec((BN, 1), lambda i: (i, 0)),
    ],
    out_shape=[
        jax.ShapeDtypeStruct((N, F), jnp.float32),
        jax.ShapeDtypeStruct((N, 1), jnp.float32),
    ],
)


def _k3_body(p0_ref, p1_ref, bias_ref, mu_ref, p_ref, sf_ref, out_ref):
    p_eff = sf_ref[0, 0] + 1.0 / (1.0 + jnp.exp(-p_ref[0, 0]))
    s = p0_ref[...] + p1_ref[...] + 1e-6
    out_ref[...] = jnp.exp(jnp.log(s) / p_eff) + mu_ref[0, 0] + bias_ref[...]


_k3 = pl.pallas_call(
    _k3_body,
    grid=(N // BN,),
    in_specs=[
        pl.BlockSpec((BN, F), lambda i: (i, 0)),
        pl.BlockSpec((BN, F), lambda i: (i, 0)),
        pl.BlockSpec((1, F), lambda i: (0, 0)),
        pl.BlockSpec((1, 1), lambda i: (0, 0), memory_space=pltpu.SMEM),
        pl.BlockSpec((1, 1), lambda i: (0, 0), memory_space=pltpu.SMEM),
        pl.BlockSpec((1, 1), lambda i: (0, 0), memory_space=pltpu.SMEM),
    ],
    out_specs=pl.BlockSpec((BN, F), lambda i: (i, 0)),
    out_shape=jax.ShapeDtypeStruct((N, F), jnp.float32),
)


_mesh = plsc.VectorSubcoreMesh(core_axis_name="c", subcore_axis_name="s")


@functools.partial(
    pl.kernel,
    mesh=_mesh,
    compiler_params=pltpu.CompilerParams(needs_layout_passes=False),
    out_type=(
        jax.ShapeDtypeStruct((E,), jnp.float32),
        jax.ShapeDtypeStruct((NC, NP), jnp.float32),
    ),
    scratch_types=[
        pltpu.VMEM((N,), jnp.float32),
        pltpu.VMEM((N,), jnp.float32),
        pltpu.VMEM((ET,), jnp.int32),
        pltpu.VMEM((ET,), jnp.int32),
        pltpu.VMEM((NB, SCH), jnp.int32),
        pltpu.VMEM((ET,), jnp.float32),
        pltpu.VMEM((L,), jnp.float32),
        pltpu.VMEM_SHARED((NP,), jnp.float32),
    ],
)
def _e1(el_hbm, er_hbm, src_hbm, dst_hbm, dst2_hbm, m_hbm, z_hbm,
        ee_hbm, den_hbm,
        el_v, er_v, src_v, dst_v, dst2_v, ee_v, m_v, den_sp):
    cid = lax.axis_index("c")
    sid = lax.axis_index("s")
    wid = sid * NC + cid
    ebase = wid * ET

    pltpu.sync_copy(el_hbm, el_v)
    pltpu.sync_copy(er_hbm, er_v)
    pltpu.sync_copy(src_hbm.at[pl.ds(ebase, ET)], src_v)
    pltpu.sync_copy(dst_hbm.at[pl.ds(ebase, ET)], dst_v)
    pltpu.sync_copy(dst2_hbm.at[wid], dst2_v)
    pltpu.sync_copy(m_hbm, m_v)
    pltpu.sync_copy(z_hbm.at[pl.ds(sid * RPT, RPT)],
                    den_sp.at[pl.ds(sid * RPT, RPT)])
    plsc.subcore_barrier()

    mvec = m_v[...]

    def body(i, carry):
        off = pl.multiple_of(i * L, 8)
        s16 = src_v[pl.ds(off, L)]
        d16 = dst_v[pl.ds(off, L)]
        e = plsc.load_gather(el_v, [s16]) + plsc.load_gather(er_v, [d16])
        e = jnp.where(e > 0, e, 0.2 * e)
        ee_v[pl.ds(off, L)] = jnp.exp(e - mvec)
        return carry

    lax.fori_loop(0, ET // L, body, 0)

    def sbody(c, carry):
        co = pl.multiple_of(c * SCH, 8)
        pltpu.sync_copy(ee_v.at[pl.ds(co, SCH)],
                        den_sp.at[dst2_v.at[c]], add=True)
        return carry

    lax.fori_loop(0, NB, sbody, 0)

    pltpu.sync_copy(ee_v, ee_hbm.at[pl.ds(ebase, ET)])
    plsc.subcore_barrier()
    pltpu.sync_copy(den_sp.at[pl.ds(sid * RPT, RPT)],
                    den_hbm.at[cid, pl.ds(sid * RPT, RPT)])


@functools.partial(
    pl.kernel,
    mesh=_mesh,
    compiler_params=pltpu.CompilerParams(needs_layout_passes=False),
    out_type=jax.ShapeDtypeStruct((NC, NP, F), jnp.float32),
    scratch_types=[
        pltpu.VMEM((EC,), jnp.int32),
        pltpu.VMEM((NCB, SCH), jnp.int32),
        pltpu.VMEM((EC,), jnp.float32),
        pltpu.VMEM((N,), jnp.float32),
        pltpu.VMEM((SCH, F), jnp.float32),
        pltpu.VMEM((SCH,), jnp.float32),
        pltpu.VMEM_SHARED((NP, F), jnp.float32),
        pltpu.SemaphoreType.DMA,
    ],
)
def _e2(pf_hbm, src_hbm, dst2_hbm, ee_hbm, inv_hbm,
        out_hbm,
        src_v, dst2_v, ee_v, inv_v, rows_v, a_v, acc_sp, sem):
    cid = lax.axis_index("c")
    sid = lax.axis_index("s")
    wid = sid * NC + cid
    ebase = wid * ET

    pltpu.sync_copy(inv_hbm, inv_v)

    zero = jnp.zeros((L,), jnp.float32)

    def zbody(r, carry):
        for c in range(F // L):
            rows_v[r, pl.ds(c * L, L)] = zero
        return carry

    lax.fori_loop(0, SCH, zbody, 0)
    for q in range(RPT // SCH):
        pltpu.sync_copy(rows_v, acc_sp.at[pl.ds(sid * RPT + q * SCH, SCH)])
    plsc.subcore_barrier()

    def chunk(ch, carry):
        pltpu.sync_copy(src_hbm.at[pl.ds(ebase + ch * EC, EC)], src_v)
        pltpu.sync_copy(dst2_hbm.at[wid * NCH + ch], dst2_v)
        pltpu.sync_copy(ee_hbm.at[pl.ds(ebase + ch * EC, EC)], ee_v)

        def batch(b, inner):
            eoff = pl.multiple_of(b * SCH, 8)
            pltpu.async_copy(pf_hbm.at[src_v.at[pl.ds(eoff, SCH)]],
                             rows_v, sem).wait()
            for g in range(SCH // L):
                d16 = dst2_v[b, pl.ds(g * L, L)]
                inv16 = plsc.load_gather(inv_v, [d16])
                a_v[pl.ds(g * L, L)] = ee_v[pl.ds(eoff + g * L, L)] * inv16

            def mul(j, mcarry):
                ab = plsc.load_gather(a_v, [jnp.full((L,), 0, jnp.int32) + j])
                for c in range(F // L):
                    rows_v[j, pl.ds(c * L, L)] = rows_v[j, pl.ds(c * L, L)] * ab
                return mcarry

            lax.fori_loop(0, SCH, mul, 0)
            pltpu.sync_copy(rows_v, acc_sp.at[dst2_v.at[b]], add=True)
            return inner

        lax.fori_loop(0, NCB, batch, 0)
        return carry

    lax.fori_loop(0, NCH, chunk, 0)
    plsc.subcore_barrier()
    pltpu.sync_copy(acc_sp.at[pl.ds(sid * RPT, RPT)],
                    out_hbm.at[cid, pl.ds(sid * RPT, RPT)])


def kernel(feat, edge_index, p, s_f, W, attn_l, attn_r, bias):
    src = edge_index[0]
    dst = edge_index[1]
    dst2 = dst.reshape(NW, NB, SCH)
    dst2b = dst.reshape(NW * NCH, NCB, SCH)
    al = attn_l.reshape(1, F)
    ar = attn_r.reshape(1, F)
    p11 = p.reshape(1, 1)
    sf11 = s_f.reshape(1, 1)

    fs, el, er, mu, ml, mr = _k1(feat, W.T, al, ar)

    shift = jnp.maximum(ml[0, 0] + mr[0, 0], 0.0)
    mvec = jnp.full((L,), shift, jnp.float32)
    zeros1 = jnp.zeros((NP,), jnp.float32)

    ee, den = _e1(el.reshape(N), er.reshape(N), src, dst, dst2, mvec, zeros1)

    pre_f, inv_d = _k2(fs, den[0, :N].reshape(N, 1), den[1, :N].reshape(N, 1),
                       mu, p11, sf11)

    parts = _e2(pre_f, src, dst2b, ee, inv_d.reshape(N))

    out = _k3(parts[0, :N], parts[1, :N], bias.reshape(1, F), mu, p11, sf11)
    return out


# speedup vs baseline: 34.7551x; 1.4208x over previous
import functools

import jax
import jax.numpy as jnp
from jax import lax
from jax.experimental import pallas as pl
from jax.experimental.pallas import tpu as pltpu
from jax.experimental.pallas import tpu_sc as plsc

N = 10000
E = 320000
F = 128

NC = 2
NS = 16
L = 16
NW = NC * NS
ET = E // NW
SCH = 80
NB = ET // SCH
EC = 2000
NCH = ET // EC
NCB = EC // SCH
NP = 10240
RPT = NP // NS

BN = 1000


def _k1_body(feat_ref, wt_ref, al_ref, ar_ref,
             fs_ref, el_ref, er_ref, mu_ref, ml_ref, mr_ref):
    i = pl.program_id(0)
    fs = jnp.dot(feat_ref[...], wt_ref[...], preferred_element_type=jnp.float32)
    fs_ref[...] = fs
    el = jnp.sum(fs * al_ref[...], axis=1)
    er = jnp.sum(fs * ar_ref[...], axis=1)
    el_ref[...] = el[:, None]
    er_ref[...] = er[:, None]
    bmin = jnp.min(fs)
    bml = jnp.max(el)
    bmr = jnp.max(er)

    @pl.when(i == 0)
    def _():
        mu_ref[0, 0] = bmin
        ml_ref[0, 0] = bml
        mr_ref[0, 0] = bmr

    @pl.when(i > 0)
    def _():
        mu_ref[0, 0] = jnp.minimum(mu_ref[0, 0], bmin)
        ml_ref[0, 0] = jnp.maximum(ml_ref[0, 0], bml)
        mr_ref[0, 0] = jnp.maximum(mr_ref[0, 0], bmr)


_k1 = pl.pallas_call(
    _k1_body,
    grid=(N // BN,),
    in_specs=[
        pl.BlockSpec((BN, F), lambda i: (i, 0)),
        pl.BlockSpec((F, F), lambda i: (0, 0)),
        pl.BlockSpec((1, F), lambda i: (0, 0)),
        pl.BlockSpec((1, F), lambda i: (0, 0)),
    ],
    out_specs=[
        pl.BlockSpec((BN, F), lambda i: (i, 0)),
        pl.BlockSpec((BN, 1), lambda i: (i, 0)),
        pl.BlockSpec((BN, 1), lambda i: (i, 0)),
        pl.BlockSpec((1, 1), lambda i: (0, 0), memory_space=pltpu.SMEM),
        pl.BlockSpec((1, 1), lambda i: (0, 0), memory_space=pltpu.SMEM),
        pl.BlockSpec((1, 1), lambda i: (0, 0), memory_space=pltpu.SMEM),
    ],
    out_shape=[
        jax.ShapeDtypeStruct((N, F), jnp.float32),
        jax.ShapeDtypeStruct((N, 1), jnp.float32),
        jax.ShapeDtypeStruct((N, 1), jnp.float32),
        jax.ShapeDtypeStruct((1, 1), jnp.float32),
        jax.ShapeDtypeStruct((1, 1), jnp.float32),
        jax.ShapeDtypeStruct((1, 1), jnp.float32),
    ],
)


def _k2_body(fs_ref, d0_ref, d1_ref, mu_ref, p_ref, sf_ref,
             pf_ref, inv_ref):
    p_eff = sf_ref[0, 0] + 1.0 / (1.0 + jnp.exp(-p_ref[0, 0]))
    x = fs_ref[...] - mu_ref[0, 0] + 1e-6
    pf_ref[...] = jnp.exp(p_eff * jnp.log(x))
    inv_ref[...] = 1.0 / (d0_ref[...] + d1_ref[...])


_k2 = pl.pallas_call(
    _k2_body,
    grid=(N // BN,),
    in_specs=[
        pl.BlockSpec((BN, F), lambda i: (i, 0)),
        pl.BlockSpec((BN, 1), lambda i: (i, 0)),
        pl.BlockSpec((BN, 1), lambda i: (i, 0)),
        pl.BlockSpec((1, 1), lambda i: (0, 0), memory_space=pltpu.SMEM),
        pl.BlockSpec((1, 1), lambda i: (0, 0), memory_space=pltpu.SMEM),
        pl.BlockSpec((1, 1), lambda i: (0, 0), memory_space=pltpu.SMEM),
    ],
    out_specs=[
        pl.BlockSpec((BN, F), lambda i: (i, 0)),
        pl.BlockSpec((BN, 1), lambda i: (i, 0)),
    ],
    out_shape=[
        jax.ShapeDtypeStruct((N, F), jnp.float32),
        jax.ShapeDtypeStruct((N, 1), jnp.float32),
    ],
)


def _k3_body(p0_ref, p1_ref, bias_ref, mu_ref, p_ref, sf_ref, out_ref):
    p_eff = sf_ref[0, 0] + 1.0 / (1.0 + jnp.exp(-p_ref[0, 0]))
    s = p0_ref[...] + p1_ref[...] + 1e-6
    out_ref[...] = jnp.exp(jnp.log(s) / p_eff) + mu_ref[0, 0] + bias_ref[...]


_k3 = pl.pallas_call(
    _k3_body,
    grid=(N // BN,),
    in_specs=[
        pl.BlockSpec((BN, F), lambda i: (i, 0)),
        pl.BlockSpec((BN, F), lambda i: (i, 0)),
        pl.BlockSpec((1, F), lambda i: (0, 0)),
        pl.BlockSpec((1, 1), lambda i: (0, 0), memory_space=pltpu.SMEM),
        pl.BlockSpec((1, 1), lambda i: (0, 0), memory_space=pltpu.SMEM),
        pl.BlockSpec((1, 1), lambda i: (0, 0), memory_space=pltpu.SMEM),
    ],
    out_specs=pl.BlockSpec((BN, F), lambda i: (i, 0)),
    out_shape=jax.ShapeDtypeStruct((N, F), jnp.float32),
)


_mesh = plsc.VectorSubcoreMesh(core_axis_name="c", subcore_axis_name="s")


@functools.partial(
    pl.kernel,
    mesh=_mesh,
    compiler_params=pltpu.CompilerParams(needs_layout_passes=False),
    out_type=(
        jax.ShapeDtypeStruct((E,), jnp.float32),
        jax.ShapeDtypeStruct((NC, NP), jnp.float32),
    ),
    scratch_types=[
        pltpu.VMEM((N,), jnp.float32),
        pltpu.VMEM((N,), jnp.float32),
        pltpu.VMEM((ET,), jnp.int32),
        pltpu.VMEM((ET,), jnp.int32),
        pltpu.VMEM((NB, SCH), jnp.int32),
        pltpu.VMEM((ET,), jnp.float32),
        pltpu.VMEM((L,), jnp.float32),
        pltpu.VMEM_SHARED((NP,), jnp.float32),
    ],
)
def _e1(el_hbm, er_hbm, src_hbm, dst_hbm, dst2_hbm, m_hbm, z_hbm,
        ee_hbm, den_hbm,
        el_v, er_v, src_v, dst_v, dst2_v, ee_v, m_v, den_sp):
    cid = lax.axis_index("c")
    sid = lax.axis_index("s")
    wid = sid * NC + cid
    ebase = wid * ET

    pltpu.sync_copy(el_hbm, el_v)
    pltpu.sync_copy(er_hbm, er_v)
    pltpu.sync_copy(src_hbm.at[pl.ds(ebase, ET)], src_v)
    pltpu.sync_copy(dst_hbm.at[pl.ds(ebase, ET)], dst_v)
    pltpu.sync_copy(dst2_hbm.at[wid], dst2_v)
    pltpu.sync_copy(m_hbm, m_v)
    pltpu.sync_copy(z_hbm.at[pl.ds(sid * RPT, RPT)],
                    den_sp.at[pl.ds(sid * RPT, RPT)])
    plsc.subcore_barrier()

    mvec = m_v[...]

    def body(i, carry):
        off = pl.multiple_of(i * L, 8)
        s16 = src_v[pl.ds(off, L)]
        d16 = dst_v[pl.ds(off, L)]
        e = plsc.load_gather(el_v, [s16]) + plsc.load_gather(er_v, [d16])
        e = jnp.where(e > 0, e, 0.2 * e)
        ee_v[pl.ds(off, L)] = jnp.exp(e - mvec)
        return carry

    lax.fori_loop(0, ET // L, body, 0)

    def sbody(c, carry):
        co = pl.multiple_of(c * SCH, 8)
        pltpu.sync_copy(ee_v.at[pl.ds(co, SCH)],
                        den_sp.at[dst2_v.at[c]], add=True)
        return carry

    lax.fori_loop(0, NB, sbody, 0)

    pltpu.sync_copy(ee_v, ee_hbm.at[pl.ds(ebase, ET)])
    plsc.subcore_barrier()
    pltpu.sync_copy(den_sp.at[pl.ds(sid * RPT, RPT)],
                    den_hbm.at[cid, pl.ds(sid * RPT, RPT)])


@functools.partial(
    pl.kernel,
    mesh=_mesh,
    compiler_params=pltpu.CompilerParams(needs_layout_passes=False),
    out_type=jax.ShapeDtypeStruct((NC, NP, F), jnp.float32),
    scratch_types=[
        pltpu.VMEM((EC,), jnp.int32),
        pltpu.VMEM((NCB, SCH), jnp.int32),
        pltpu.VMEM((EC,), jnp.float32),
        pltpu.VMEM((N,), jnp.float32),
        pltpu.VMEM((SCH, F), jnp.float32),
        pltpu.VMEM((SCH, F), jnp.float32),
        pltpu.VMEM_SHARED((NP, F), jnp.float32),
        pltpu.SemaphoreType.DMA,
        pltpu.SemaphoreType.DMA,
    ],
)
def _e2(pf_hbm, src_hbm, dst2_hbm, ee_hbm, inv_hbm,
        out_hbm,
        src_v, dst2_v, ee_v, inv_v, rows0_v, rows1_v, acc_sp, sem0, sem1):
    cid = lax.axis_index("c")
    sid = lax.axis_index("s")
    wid = sid * NC + cid
    ebase = wid * ET

    pltpu.sync_copy(inv_hbm, inv_v)

    zero = jnp.zeros((L,), jnp.float32)

    def zbody(r, carry):
        for c in range(F // L):
            rows0_v[r, pl.ds(c * L, L)] = zero
        return carry

    lax.fori_loop(0, SCH, zbody, 0)
    for q in range(RPT // SCH):
        pltpu.sync_copy(rows0_v, acc_sp.at[pl.ds(sid * RPT + q * SCH, SCH)])
    plsc.subcore_barrier()

    def gather(b, buf, sem):
        pltpu.async_copy(
            pf_hbm.at[src_v.at[pl.ds(pl.multiple_of(b * SCH, 8), SCH)]],
            buf, sem)

    def gwait(b, buf, sem):
        pltpu.make_async_copy(
            pf_hbm.at[src_v.at[pl.ds(pl.multiple_of(b * SCH, 8), SCH)]],
            buf, sem).wait()

    def compute_scatter(b, buf):
        eoff = b * SCH

        def mul4(j4, mcarry):
            jb = j4 * 4
            for u in range(4):
                ab = plsc.load_gather(
                    ee_v, [jnp.full((L,), 0, jnp.int32) + (eoff + jb + u)])
                for c in range(F // L):
                    buf[jb + u, pl.ds(c * L, L)] = buf[jb + u, pl.ds(c * L, L)] * ab
            return mcarry

        lax.fori_loop(0, SCH // 4, mul4, 0)
        pltpu.sync_copy(buf, acc_sp.at[dst2_v.at[b]], add=True)

    def chunk(ch, carry):
        pltpu.sync_copy(src_hbm.at[pl.ds(ebase + ch * EC, EC)], src_v)
        pltpu.sync_copy(dst2_hbm.at[wid * NCH + ch], dst2_v)
        pltpu.sync_copy(ee_hbm.at[pl.ds(ebase + ch * EC, EC)], ee_v)

        def apre(b, inner):
            eoff = pl.multiple_of(b * SCH, 8)
            for g in range(SCH // L):
                d16 = dst2_v[b, pl.ds(g * L, L)]
                inv16 = plsc.load_gather(inv_v, [d16])
                o = eoff + g * L
                ee_v[pl.ds(o, L)] = ee_v[pl.ds(o, L)] * inv16
            return inner

        lax.fori_loop(0, NCB, apre, 0)

        gather(0, rows0_v, sem0)

        def pair(k, inner):
            b0 = 2 * k
            b1 = 2 * k + 1
            gwait(b0, rows0_v, sem0)
            gather(b1, rows1_v, sem1)
            compute_scatter(b0, rows0_v)
            gwait(b1, rows1_v, sem1)
            gather(b1 + 1, rows0_v, sem0)
            compute_scatter(b1, rows1_v)
            return inner

        lax.fori_loop(0, (NCB - 1) // 2, pair, 0)
        gwait(NCB - 1, rows0_v, sem0)
        compute_scatter(NCB - 1, rows0_v)
        return carry

    lax.fori_loop(0, NCH, chunk, 0)
    plsc.subcore_barrier()
    pltpu.sync_copy(acc_sp.at[pl.ds(sid * RPT, RPT)],
                    out_hbm.at[cid, pl.ds(sid * RPT, RPT)])


def kernel(feat, edge_index, p, s_f, W, attn_l, attn_r, bias):
    src = edge_index[0]
    dst = edge_index[1]
    dst2 = dst.reshape(NW, NB, SCH)
    dst2b = dst.reshape(NW * NCH, NCB, SCH)
    al = attn_l.reshape(1, F)
    ar = attn_r.reshape(1, F)
    p11 = p.reshape(1, 1)
    sf11 = s_f.reshape(1, 1)

    fs, el, er, mu, ml, mr = _k1(feat, W.T, al, ar)

    shift = jnp.maximum(ml[0, 0] + mr[0, 0], 0.0)
    mvec = jnp.full((L,), shift, jnp.float32)
    zeros1 = jnp.zeros((NP,), jnp.float32)

    ee, den = _e1(el.reshape(N), er.reshape(N), src, dst, dst2, mvec, zeros1)

    pre_f, inv_d = _k2(fs, den[0, :N].reshape(N, 1), den[1, :N].reshape(N, 1),
                       mu, p11, sf11)

    parts = _e2(pre_f, src, dst2b, ee, inv_d.reshape(N))

    out = _k3(parts[0, :N], parts[1, :N], bias.reshape(1, F), mu, p11, sf11)
    return out
